# baseline (speedup 1.0000x reference)
import functools

import jax
import jax.numpy as jnp
from jax import lax
from jax.experimental import pallas as pl
from jax.experimental.pallas import tpu as pltpu
from jax.experimental.pallas import tpu_sc as plsc

NC = 2
NS = 16
NW = NC * NS
LANES = 16


_DN_T = (((0,), (1,)), ((), ()))
_TN_T = (((0,), (0,)), ((), ()))


def _tc_msg0_body(x_ref, w_ref, bcol_ref, o_ref):
    acc = lax.dot_general(w_ref[...], x_ref[...], _DN_T,
                          preferred_element_type=jnp.float32)
    o_ref[...] = jnp.maximum(acc + bcol_ref[...], 0.0)


def _tc_msg0(x, w, b):
    n, _ = x.shape
    dout = w.shape[1]
    return pl.pallas_call(
        _tc_msg0_body,
        out_shape=jax.ShapeDtypeStruct((dout, n), jnp.float32),
    )(x, w, b.reshape(dout, 1))


def _tc_mid_body(h_ref, at_ref, wfc_ref, bfc_ref, wp_ref, bpcol_ref,
                 h1_ref, m1_ref):
    din = h_ref.shape[1]
    top = jnp.dot(h_ref[...], wfc_ref[0:din, :],
                  preferred_element_type=jnp.float32)
    bot = lax.dot_general(at_ref[...], wfc_ref[din:2 * din, :], _TN_T,
                          preferred_element_type=jnp.float32)
    h1 = jnp.maximum(top + bot + bfc_ref[...], 0.0)
    h1_ref[...] = h1
    m1 = lax.dot_general(wp_ref[...], h1, _DN_T,
                         preferred_element_type=jnp.float32)
    m1_ref[...] = jnp.maximum(m1 + bpcol_ref[...], 0.0)


def _tc_mid(h, aggT, wfc, bfc, wp, bp):
    n, _ = h.shape
    dout = wfc.shape[1]
    dp = wp.shape[1]
    return pl.pallas_call(
        _tc_mid_body,
        out_shape=(
            jax.ShapeDtypeStruct((n, dout), jnp.float32),
            jax.ShapeDtypeStruct((dp, n), jnp.float32),
        ),
    )(h, aggT, wfc, bfc.reshape(1, dout), wp, bp.reshape(dp, 1))


def _tc_out_body(h_ref, at_ref, wfc_ref, bfc_ref, o_ref):
    din = h_ref.shape[1]
    top = jnp.dot(h_ref[...], wfc_ref[0:din, :],
                  preferred_element_type=jnp.float32)
    bot = lax.dot_general(at_ref[...], wfc_ref[din:2 * din, :], _TN_T,
                          preferred_element_type=jnp.float32)
    o_ref[...] = top + bot + bfc_ref[...]


def _tc_out(h, aggT, wfc, bfc):
    n, _ = h.shape
    dout = wfc.shape[1]
    return pl.pallas_call(
        _tc_out_body,
        out_shape=jax.ShapeDtypeStruct((n, dout), jnp.float32),
    )(h, aggT, wfc, bfc.reshape(1, dout))


UNROLL = 4


def _sc_segmax_body(n, e, chunk,
                    msg_hbm, src_hbm, dst_hbm, agg_hbm,
                    m0, m1, m2, m3,
                    a0, a1, a2, a3, b0, b1, b2, b3,
                    src0, dst0, src1, dst1, sem0, sem1):
    c = lax.axis_index("c")
    s = lax.axis_index("s")
    wid = s * NC + c

    msgs = [m0, m1, m2, m3]
    aggs = [[a0, a1, a2, a3], [b0, b1, b2, b3]]
    ebufs = [(src0, dst0, sem0), (src1, dst1, sem1)]

    for f in range(4):
        pltpu.sync_copy(msg_hbm.at[wid, f], msgs[f])

    zero = jnp.zeros((LANES,), jnp.float32)

    def zbody(i, carry):
        for p in range(2):
            for f in range(4):
                aggs[p][f][pl.ds(i * LANES, LANES)] = zero
        return carry

    lax.fori_loop(0, n // LANES, zbody, 0)

    nchunks = e // chunk

    def start_fetch(ci, buf):
        sb, db, sem = buf
        pltpu.async_copy(src_hbm.at[pl.ds(ci * chunk, chunk)], sb, sem)
        pltpu.async_copy(dst_hbm.at[pl.ds(ci * chunk, chunk)], db, sem)

    def drain(buf):
        sb, db, sem = buf
        pltpu.make_async_copy(src_hbm.at[pl.ds(0, chunk)], sb, sem).wait()
        pltpu.make_async_copy(dst_hbm.at[pl.ds(0, chunk)], db, sem).wait()

    def process(buf):
        sb, db, _ = buf

        def vec_body(g, carry2):
            base = g * (UNROLL * LANES)
            svs = [sb[pl.ds(base + u * LANES, LANES)] for u in range(UNROLL)]
            dvs = [db[pl.ds(base + u * LANES, LANES)] for u in range(UNROLL)]

            valss = [[plsc.load_gather(msgs[f], [svs[u]]) for f in range(4)]
                     for u in range(UNROLL)]
            dupmask = None
            for u in range(UNROLL):
                counts, _ = plsc.scan_count(dvs[u])
                dup = counts > 0
                dupmask = dup if dupmask is None else (dupmask | dup)
            for pair in range(UNROLL // 2):
                us = (2 * pair, 2 * pair + 1)
                curss = {u: [plsc.load_gather(aggs[u % 2][f], [dvs[u]])
                             for f in range(4)] for u in us}
                for u in us:
                    for f in range(4):
                        plsc.store_scatter(aggs[u % 2][f], [dvs[u]],
                                           valss[u][f],
                                           mask=valss[u][f] > curss[u][f])

            ndup = plsc.all_reduce_population_count(dupmask)

            @pl.when(ndup[0] > 0)
            def _():
                for u in range(UNROLL):
                    for f in range(4):
                        def fix_body(_, u=u, f=f):
                            acc = aggs[u % 2][f]
                            cur = plsc.load_gather(acc, [dvs[u]])
                            plsc.store_scatter(acc, [dvs[u]],
                                               valss[u][f],
                                               mask=valss[u][f] > cur)
                            cur2 = plsc.load_gather(acc, [dvs[u]])
                            pend = plsc.all_reduce_population_count(
                                valss[u][f] > cur2)
                            return pend[0] > 0

                        lax.while_loop(lambda keep: keep, fix_body,
                                       jnp.bool_(True))

            return carry2

        lax.fori_loop(0, chunk // (UNROLL * LANES), vec_body, 0)

    start_fetch(0, ebufs[0])

    def pair_body(i, carry):
        start_fetch(2 * i + 1, ebufs[1])
        drain(ebufs[0])
        process(ebufs[0])

        @pl.when(2 * i + 2 < nchunks)
        def _():
            start_fetch(2 * i + 2, ebufs[0])

        drain(ebufs[1])
        process(ebufs[1])
        return carry

    lax.fori_loop(0, nchunks // 2, pair_body, 0)

    def merge_body(i, carry):
        sl = pl.ds(i * LANES, LANES)
        for f in range(4):
            aggs[0][f][sl] = jnp.maximum(aggs[0][f][sl], aggs[1][f][sl])
        return carry

    lax.fori_loop(0, n // LANES, merge_body, 0)

    for f in range(4):
        pltpu.sync_copy(aggs[0][f], agg_hbm.at[wid, f])


def _sc_segmax(msg_t, src, dst, chunk):
    nw, nf, n = msg_t.shape
    e = src.shape[0]
    assert nw == NW and nf == 4 and e % (2 * chunk) == 0
    assert chunk % (UNROLL * LANES) == 0

    mesh = plsc.VectorSubcoreMesh(core_axis_name="c", subcore_axis_name="s",
                                  num_cores=NC, num_subcores=NS)
    kern = pl.kernel(
        functools.partial(_sc_segmax_body, n, e, chunk),
        out_type=jax.ShapeDtypeStruct((NW, 4, n), jnp.float32),
        mesh=mesh,
        compiler_params=pltpu.CompilerParams(needs_layout_passes=False),
        scratch_types=(
            [pltpu.VMEM((n,), jnp.float32) for _ in range(12)]
            + [pltpu.VMEM((chunk,), jnp.int32) for _ in range(4)]
            + [pltpu.SemaphoreType.DMA, pltpu.SemaphoreType.DMA]
        ),
    )
    return kern(msg_t, src, dst)


def kernel(x, edge_index, W_pool0, b_pool0, W_pool1, b_pool1,
           W_fc0, b_fc0, W_fc1, b_fc1):
    n, d = x.shape
    src = edge_index[0].astype(jnp.int32)
    dst = edge_index[1].astype(jnp.int32)
    e = src.shape[0]
    chunk = 1600

    msg0T = _tc_msg0(x, W_pool0, b_pool0)
    agg0_t = _sc_segmax(msg0T.reshape(NW, d // NW, n), src, dst, chunk)
    agg0T = agg0_t.reshape(d, n)

    h1, msg1T = _tc_mid(x, agg0T, W_fc0, b_fc0, W_pool1, b_pool1)

    d1 = msg1T.shape[0]
    agg1_t = _sc_segmax(msg1T.reshape(NW, d1 // NW, n), src, dst, chunk)
    agg1T = agg1_t.reshape(d1, n)

    return _tc_out(h1, agg1T, W_fc1, b_fc1)


# speedup vs baseline: 5.5961x; 5.5961x over previous
import functools

import jax
import jax.numpy as jnp
from jax import lax
from jax.experimental import pallas as pl
from jax.experimental.pallas import tpu as pltpu
from jax.experimental.pallas import tpu_sc as plsc

NC = 2
NS = 16
NW = NC * NS
LANES = 16


_DN_T = (((0,), (1,)), ((), ()))
_TN_T = (((0,), (0,)), ((), ()))


def _tc_msg0_body(x_ref, w_ref, bcol_ref, o_ref):
    acc = lax.dot_general(w_ref[...], x_ref[...], _DN_T,
                          preferred_element_type=jnp.float32)
    o_ref[...] = jnp.maximum(acc + bcol_ref[...], 0.0)


def _tc_msg0(x, w, b):
    n, _ = x.shape
    dout = w.shape[1]
    return pl.pallas_call(
        _tc_msg0_body,
        out_shape=jax.ShapeDtypeStruct((dout, n), jnp.float32),
    )(x, w, b.reshape(dout, 1))


def _tc_mid_body(h_ref, at_ref, wfc_ref, bfc_ref, wp_ref, bpcol_ref,
                 h1_ref, m1_ref):
    din = h_ref.shape[1]
    top = jnp.dot(h_ref[...], wfc_ref[0:din, :],
                  preferred_element_type=jnp.float32)
    bot = lax.dot_general(at_ref[...], wfc_ref[din:2 * din, :], _TN_T,
                          preferred_element_type=jnp.float32)
    h1 = jnp.maximum(top + bot + bfc_ref[...], 0.0)
    h1_ref[...] = h1
    m1 = lax.dot_general(wp_ref[...], h1, _DN_T,
                         preferred_element_type=jnp.float32)
    m1_ref[...] = jnp.maximum(m1 + bpcol_ref[...], 0.0)


def _tc_mid(h, aggT, wfc, bfc, wp, bp):
    n, _ = h.shape
    dout = wfc.shape[1]
    dp = wp.shape[1]
    return pl.pallas_call(
        _tc_mid_body,
        out_shape=(
            jax.ShapeDtypeStruct((n, dout), jnp.float32),
            jax.ShapeDtypeStruct((dp, n), jnp.float32),
        ),
    )(h, aggT, wfc, bfc.reshape(1, dout), wp, bp.reshape(dp, 1))


def _tc_out_body(h_ref, at_ref, wfc_ref, bfc_ref, o_ref):
    din = h_ref.shape[1]
    top = jnp.dot(h_ref[...], wfc_ref[0:din, :],
                  preferred_element_type=jnp.float32)
    bot = lax.dot_general(at_ref[...], wfc_ref[din:2 * din, :], _TN_T,
                          preferred_element_type=jnp.float32)
    o_ref[...] = top + bot + bfc_ref[...]


def _tc_out(h, aggT, wfc, bfc):
    n, _ = h.shape
    dout = wfc.shape[1]
    return pl.pallas_call(
        _tc_out_body,
        out_shape=jax.ShapeDtypeStruct((n, dout), jnp.float32),
    )(h, aggT, wfc, bfc.reshape(1, dout))


UNROLL = 4


def _sc_segmax_body(n, e, chunk,
                    msg_hbm, src_hbm, dst_hbm, agg_hbm,
                    m0, m1, m2, m3,
                    a0, a1, a2, a3, b0, b1, b2, b3,
                    src0, dst0, src1, dst1, sem0, sem1):
    c = lax.axis_index("c")
    s = lax.axis_index("s")
    wid = s * NC + c

    ubase, _ = plsc.scan_count(lax.iota(jnp.int32, 16))

    msgs = [m0, m1, m2, m3]
    aggs = [[a0, a1, a2, a3], [b0, b1, b2, b3]]
    ebufs = [(src0, dst0, sem0), (src1, dst1, sem1)]

    for f in range(4):
        pltpu.sync_copy(msg_hbm.at[wid, f], msgs[f])

    zero = jnp.zeros((LANES,), jnp.float32)

    def zbody(i, carry):
        for p in range(2):
            for f in range(4):
                aggs[p][f][pl.ds(i * LANES, LANES)] = zero
        return carry

    lax.fori_loop(0, n // LANES, zbody, 0)

    nchunks = e // chunk

    def start_fetch(ci, buf):
        sb, db, sem = buf
        pltpu.async_copy(src_hbm.at[pl.ds(ci * chunk, chunk)], sb, sem)
        pltpu.async_copy(dst_hbm.at[pl.ds(ci * chunk, chunk)], db, sem)

    def drain(buf):
        sb, db, sem = buf
        pltpu.make_async_copy(src_hbm.at[pl.ds(0, chunk)], sb, sem).wait()
        pltpu.make_async_copy(dst_hbm.at[pl.ds(0, chunk)], db, sem).wait()

    def process(buf):
        sb, db, _ = buf

        def vec_body(g, carry2):
            base = g * (UNROLL * LANES)
            svs = [sb[pl.ds(base + u * LANES, LANES)] for u in range(UNROLL)]
            dvs = [db[pl.ds(base + u * LANES, LANES)] for u in range(UNROLL)]

            valss = [[plsc.load_gather(msgs[f], [svs[u]]) for f in range(4)]
                     for u in range(UNROLL)]
            dupmask = None
            for u in range(UNROLL):
                counts, _ = plsc.scan_count(dvs[u])
                dup = counts > ubase
                dupmask = dup if dupmask is None else (dupmask | dup)
            for pair in range(UNROLL // 2):
                us = (2 * pair, 2 * pair + 1)
                curss = {u: [plsc.load_gather(aggs[u % 2][f], [dvs[u]])
                             for f in range(4)] for u in us}
                for u in us:
                    for f in range(4):
                        plsc.store_scatter(aggs[u % 2][f], [dvs[u]],
                                           valss[u][f],
                                           mask=valss[u][f] > curss[u][f])

            ndup = plsc.all_reduce_population_count(dupmask)

            @pl.when(ndup[0] > 0)
            def _():
                for u in range(UNROLL):
                    for f in range(4):
                        def fix_body(_, u=u, f=f):
                            acc = aggs[u % 2][f]
                            cur = plsc.load_gather(acc, [dvs[u]])
                            plsc.store_scatter(acc, [dvs[u]],
                                               valss[u][f],
                                               mask=valss[u][f] > cur)
                            cur2 = plsc.load_gather(acc, [dvs[u]])
                            pend = plsc.all_reduce_population_count(
                                valss[u][f] > cur2)
                            return pend[0] > 0

                        lax.while_loop(lambda keep: keep, fix_body,
                                       jnp.bool_(True))

            return carry2

        lax.fori_loop(0, chunk // (UNROLL * LANES), vec_body, 0)

    start_fetch(0, ebufs[0])

    def pair_body(i, carry):
        start_fetch(2 * i + 1, ebufs[1])
        drain(ebufs[0])
        process(ebufs[0])

        @pl.when(2 * i + 2 < nchunks)
        def _():
            start_fetch(2 * i + 2, ebufs[0])

        drain(ebufs[1])
        process(ebufs[1])
        return carry

    lax.fori_loop(0, nchunks // 2, pair_body, 0)

    def merge_body(i, carry):
        sl = pl.ds(i * LANES, LANES)
        for f in range(4):
            aggs[0][f][sl] = jnp.maximum(aggs[0][f][sl], aggs[1][f][sl])
        return carry

    lax.fori_loop(0, n // LANES, merge_body, 0)

    for f in range(4):
        pltpu.sync_copy(aggs[0][f], agg_hbm.at[wid, f])


def _sc_segmax(msg_t, src, dst, chunk):
    nw, nf, n = msg_t.shape
    e = src.shape[0]
    assert nw == NW and nf == 4 and e % (2 * chunk) == 0
    assert chunk % (UNROLL * LANES) == 0

    mesh = plsc.VectorSubcoreMesh(core_axis_name="c", subcore_axis_name="s",
                                  num_cores=NC, num_subcores=NS)
    kern = pl.kernel(
        functools.partial(_sc_segmax_body, n, e, chunk),
        out_type=jax.ShapeDtypeStruct((NW, 4, n), jnp.float32),
        mesh=mesh,
        compiler_params=pltpu.CompilerParams(needs_layout_passes=False),
        scratch_types=(
            [pltpu.VMEM((n,), jnp.float32) for _ in range(12)]
            + [pltpu.VMEM((chunk,), jnp.int32) for _ in range(4)]
            + [pltpu.SemaphoreType.DMA, pltpu.SemaphoreType.DMA]
        ),
    )
    return kern(msg_t, src, dst)


def kernel(x, edge_index, W_pool0, b_pool0, W_pool1, b_pool1,
           W_fc0, b_fc0, W_fc1, b_fc1):
    n, d = x.shape
    src = edge_index[0].astype(jnp.int32)
    dst = edge_index[1].astype(jnp.int32)
    e = src.shape[0]
    chunk = 1600

    msg0T = _tc_msg0(x, W_pool0, b_pool0)
    agg0_t = _sc_segmax(msg0T.reshape(NW, d // NW, n), src, dst, chunk)
    agg0T = agg0_t.reshape(d, n)

    h1, msg1T = _tc_mid(x, agg0T, W_fc0, b_fc0, W_pool1, b_pool1)

    d1 = msg1T.shape[0]
    agg1_t = _sc_segmax(msg1T.reshape(NW, d1 // NW, n), src, dst, chunk)
    agg1T = agg1_t.reshape(d1, n)

    return _tc_out(h1, agg1T, W_fc1, b_fc1)


# speedup vs baseline: 7.7542x; 1.3856x over previous
import functools

import jax
import jax.numpy as jnp
from jax import lax
from jax.experimental import pallas as pl
from jax.experimental.pallas import tpu as pltpu
from jax.experimental.pallas import tpu_sc as plsc

NC = 2
NS = 16
NW = NC * NS
LANES = 16
UNROLL = 4

_DN_T = (((0,), (1,)), ((), ()))
_TN_T = (((0,), (0,)), ((), ()))


def _pack_pair(lo_f32, hi_f32):
    lo = lax.bitcast_convert_type(lo_f32.astype(jnp.bfloat16),
                                  jnp.uint16).astype(jnp.uint32)
    hi = lax.bitcast_convert_type(hi_f32.astype(jnp.bfloat16),
                                  jnp.uint16).astype(jnp.uint32)
    return lax.bitcast_convert_type(lo | (hi << 16), jnp.int32)


def _unpack_pair(packed):
    w = lax.bitcast_convert_type(packed, jnp.uint32)
    lo = lax.bitcast_convert_type((w & 0xFFFF).astype(jnp.uint16),
                                  jnp.bfloat16).astype(jnp.float32)
    hi = lax.bitcast_convert_type((w >> 16).astype(jnp.uint16),
                                  jnp.bfloat16).astype(jnp.float32)
    return lo, hi


def _tc_msg0_body(x_ref, we_ref, wo_ref, be_ref, bo_ref, o_ref):
    ev = jnp.maximum(
        lax.dot_general(we_ref[...], x_ref[...], _DN_T,
                        preferred_element_type=jnp.float32) + be_ref[...],
        0.0)
    od = jnp.maximum(
        lax.dot_general(wo_ref[...], x_ref[...], _DN_T,
                        preferred_element_type=jnp.float32) + bo_ref[...],
        0.0)
    o_ref[...] = _pack_pair(ev, od)


def _tc_msg0(x, we, wo, be, bo):
    n = x.shape[0]
    k = we.shape[1]
    return pl.pallas_call(
        _tc_msg0_body,
        out_shape=jax.ShapeDtypeStruct((k, n), jnp.int32),
    )(x, we, wo, be.reshape(k, 1), bo.reshape(k, 1))


def _tc_mid_body(h_ref, ap_ref, wbe_ref, wbo_ref, wt_ref, bfc_ref,
                 wpe_ref, wpo_ref, bpe_ref, bpo_ref, h1_ref, m1_ref):
    age, ago = _unpack_pair(ap_ref[...])
    top = jnp.dot(h_ref[...], wt_ref[...],
                  preferred_element_type=jnp.float32)
    bot = (lax.dot_general(age, wbe_ref[...], _TN_T,
                           preferred_element_type=jnp.float32)
           + lax.dot_general(ago, wbo_ref[...], _TN_T,
                             preferred_element_type=jnp.float32))
    h1 = jnp.maximum(top + bot + bfc_ref[...], 0.0)
    h1_ref[...] = h1
    ev = jnp.maximum(
        lax.dot_general(wpe_ref[...], h1, _DN_T,
                        preferred_element_type=jnp.float32) + bpe_ref[...],
        0.0)
    od = jnp.maximum(
        lax.dot_general(wpo_ref[...], h1, _DN_T,
                        preferred_element_type=jnp.float32) + bpo_ref[...],
        0.0)
    m1_ref[...] = _pack_pair(ev, od)


def _tc_mid(h, aggP, wbe, wbo, wt, bfc, wpe, wpo, bpe, bpo):
    n = h.shape[0]
    dout = wt.shape[1]
    k = wpe.shape[1]
    return pl.pallas_call(
        _tc_mid_body,
        out_shape=(
            jax.ShapeDtypeStruct((n, dout), jnp.float32),
            jax.ShapeDtypeStruct((k, n), jnp.int32),
        ),
    )(h, aggP, wbe, wbo, wt, bfc.reshape(1, dout),
      wpe, wpo, bpe.reshape(k, 1), bpo.reshape(k, 1))


def _tc_out_body(h_ref, ap_ref, wbe_ref, wbo_ref, wt_ref, bfc_ref, o_ref):
    age, ago = _unpack_pair(ap_ref[...])
    top = jnp.dot(h_ref[...], wt_ref[...],
                  preferred_element_type=jnp.float32)
    bot = (lax.dot_general(age, wbe_ref[...], _TN_T,
                           preferred_element_type=jnp.float32)
           + lax.dot_general(ago, wbo_ref[...], _TN_T,
                             preferred_element_type=jnp.float32))
    o_ref[...] = top + bot + bfc_ref[...]


def _tc_out(h, aggP, wbe, wbo, wt, bfc):
    n = h.shape[0]
    dout = wt.shape[1]
    return pl.pallas_call(
        _tc_out_body,
        out_shape=jax.ShapeDtypeStruct((n, dout), jnp.float32),
    )(h, aggP, wbe, wbo, wt, bfc.reshape(1, dout))


def _tc_pack_edges_body(s_ref, d_ref, o_ref):
    o_ref[...] = s_ref[...] | (d_ref[...] << 16)


def _tc_pack_edges(src, dst, rows=2500):
    e = src.shape[0]
    packed = pl.pallas_call(
        _tc_pack_edges_body,
        out_shape=jax.ShapeDtypeStruct((rows, e // rows), jnp.int32),
    )(src.reshape(rows, e // rows), dst.reshape(rows, e // rows))
    return packed.reshape(e)


def _pmax(cur_w, val_w):
    cur_b = plsc.bitcast(cur_w, jnp.bfloat16)
    val_b = plsc.bitcast(val_w, jnp.bfloat16)
    return plsc.bitcast(jnp.maximum(cur_b, val_b), jnp.int32)


def _sc_segmax_body(n, e, chunk,
                    msg_hbm, edges_hbm, agg_hbm,
                    m0, m1,
                    a00, a01, a10, a11, a20, a21, a30, a31,
                    eb0, eb1, sem0, sem1):
    c = lax.axis_index("c")
    s = lax.axis_index("s")
    wid = s * NC + c

    ubase, _ = plsc.scan_count(lax.iota(jnp.int32, 16))

    msgs = [m0, m1]
    aggs = [[a00, a01], [a10, a11], [a20, a21], [a30, a31]]
    ebufs = [(eb0, sem0), (eb1, sem1)]

    for f in range(2):
        pltpu.sync_copy(msg_hbm.at[wid, f], msgs[f])

    zero = jnp.zeros((LANES,), jnp.int32)

    def zbody(i, carry):
        for u in range(UNROLL):
            for f in range(2):
                aggs[u][f][pl.ds(i * LANES, LANES)] = zero
        return carry

    lax.fori_loop(0, n // LANES, zbody, 0)

    nchunks = e // chunk

    def start_fetch(ci, buf):
        eb, sem = buf
        pltpu.async_copy(edges_hbm.at[pl.ds(ci * chunk, chunk)], eb, sem)

    def drain(buf):
        eb, sem = buf
        pltpu.make_async_copy(edges_hbm.at[pl.ds(0, chunk)], eb, sem).wait()

    def process(buf):
        eb, _ = buf

        def vec_body(g, carry2):
            base = g * (UNROLL * LANES)
            ws = [eb[pl.ds(base + u * LANES, LANES)] for u in range(UNROLL)]
            svs = [w & 0xFFFF for w in ws]
            dvs = [lax.shift_right_logical(w, 16) for w in ws]

            valss = [[plsc.load_gather(msgs[f], [svs[u]]) for f in range(2)]
                     for u in range(UNROLL)]
            dupmask = None
            for u in range(UNROLL):
                counts, _ = plsc.scan_count(dvs[u])
                dup = counts > ubase
                dupmask = dup if dupmask is None else (dupmask | dup)
            curss = [[plsc.load_gather(aggs[u][f], [dvs[u]])
                      for f in range(2)] for u in range(UNROLL)]
            for u in range(UNROLL):
                for f in range(2):
                    new_w = _pmax(curss[u][f], valss[u][f])
                    plsc.store_scatter(aggs[u][f], [dvs[u]], new_w,
                                       mask=new_w != curss[u][f])

            ndup = plsc.all_reduce_population_count(dupmask)

            @pl.when(ndup[0] > 0)
            def _():
                for u in range(UNROLL):
                    for f in range(2):
                        def fix_body(_, u=u, f=f):
                            acc = aggs[u][f]
                            cur = plsc.load_gather(acc, [dvs[u]])
                            new_w = _pmax(cur, valss[u][f])
                            plsc.store_scatter(acc, [dvs[u]], new_w,
                                               mask=new_w != cur)
                            cur2 = plsc.load_gather(acc, [dvs[u]])
                            new2 = _pmax(cur2, valss[u][f])
                            pend = plsc.all_reduce_population_count(
                                new2 != cur2)
                            return pend[0] > 0

                        lax.while_loop(lambda keep: keep, fix_body,
                                       jnp.bool_(True))

            return carry2

        lax.fori_loop(0, chunk // (UNROLL * LANES), vec_body, 0)

    start_fetch(0, ebufs[0])

    def pair_body(i, carry):
        start_fetch(2 * i + 1, ebufs[1])
        drain(ebufs[0])
        process(ebufs[0])

        @pl.when(2 * i + 2 < nchunks)
        def _():
            start_fetch(2 * i + 2, ebufs[0])

        drain(ebufs[1])
        process(ebufs[1])
        return carry

    lax.fori_loop(0, nchunks // 2, pair_body, 0)

    def merge_body(i, carry):
        sl = pl.ds(i * LANES, LANES)
        for f in range(2):
            m01 = _pmax(aggs[0][f][sl], aggs[1][f][sl])
            m23 = _pmax(aggs[2][f][sl], aggs[3][f][sl])
            aggs[0][f][sl] = _pmax(m01, m23)
        return carry

    lax.fori_loop(0, n // LANES, merge_body, 0)

    for f in range(2):
        pltpu.sync_copy(aggs[0][f], agg_hbm.at[wid, f])


def _sc_segmax(msgP_t, epacked, chunk):
    nw, nf, n = msgP_t.shape
    e = epacked.shape[0]
    assert nw == NW and nf == 2 and e % (2 * chunk) == 0
    assert chunk % (UNROLL * LANES) == 0

    mesh = plsc.VectorSubcoreMesh(core_axis_name="c", subcore_axis_name="s",
                                  num_cores=NC, num_subcores=NS)
    kern = pl.kernel(
        functools.partial(_sc_segmax_body, n, e, chunk),
        out_type=jax.ShapeDtypeStruct((NW, 2, n), jnp.int32),
        mesh=mesh,
        compiler_params=pltpu.CompilerParams(needs_layout_passes=False),
        scratch_types=(
            [pltpu.VMEM((n,), jnp.int32) for _ in range(10)]
            + [pltpu.VMEM((chunk,), jnp.int32) for _ in range(2)]
            + [pltpu.SemaphoreType.DMA, pltpu.SemaphoreType.DMA]
        ),
    )
    return kern(msgP_t, epacked)


def kernel(x, edge_index, W_pool0, b_pool0, W_pool1, b_pool1,
           W_fc0, b_fc0, W_fc1, b_fc1):
    n, d = x.shape
    src = edge_index[0].astype(jnp.int32)
    dst = edge_index[1].astype(jnp.int32)
    chunk = 3200
    epacked = _tc_pack_edges(src, dst)

    wp0e, wp0o = W_pool0[:, 0::2], W_pool0[:, 1::2]
    bp0e, bp0o = b_pool0[0::2], b_pool0[1::2]
    wp1e, wp1o = W_pool1[:, 0::2], W_pool1[:, 1::2]
    bp1e, bp1o = b_pool1[0::2], b_pool1[1::2]
    wfc0t, wfc0b = W_fc0[:d], W_fc0[d:]
    wfc0be, wfc0bo = wfc0b[0::2], wfc0b[1::2]
    d1 = W_fc0.shape[1]
    wfc1t, wfc1b = W_fc1[:d1], W_fc1[d1:]
    wfc1be, wfc1bo = wfc1b[0::2], wfc1b[1::2]

    msg0P = _tc_msg0(x, wp0e, wp0o, bp0e, bp0o)
    agg0P = _sc_segmax(msg0P.reshape(NW, 2, n), epacked, chunk)

    h1, msg1P = _tc_mid(x, agg0P.reshape(d // 2, n), wfc0be, wfc0bo,
                        wfc0t, b_fc0, wp1e, wp1o, bp1e, bp1o)

    agg1P = _sc_segmax(msg1P.reshape(NW, 2, n), epacked, chunk)

    return _tc_out(h1, agg1P.reshape(d1 // 2, n), wfc1be, wfc1bo,
                   wfc1t, b_fc1)


# speedup vs baseline: 7.8565x; 1.0132x over previous
import functools

import jax
import jax.numpy as jnp
from jax import lax
from jax.experimental import pallas as pl
from jax.experimental.pallas import tpu as pltpu
from jax.experimental.pallas import tpu_sc as plsc

NC = 2
NS = 16
NW = NC * NS
LANES = 16
UNROLL = 4

_DN_T = (((0,), (1,)), ((), ()))
_TN_T = (((0,), (0,)), ((), ()))


def _pack_pair(lo_f32, hi_f32):
    lo = lax.bitcast_convert_type(lo_f32.astype(jnp.bfloat16),
                                  jnp.uint16).astype(jnp.uint32)
    hi = lax.bitcast_convert_type(hi_f32.astype(jnp.bfloat16),
                                  jnp.uint16).astype(jnp.uint32)
    return lax.bitcast_convert_type(lo | (hi << 16), jnp.int32)


def _unpack_pair(packed):
    w = lax.bitcast_convert_type(packed, jnp.uint32)
    lo = lax.bitcast_convert_type((w & 0xFFFF).astype(jnp.uint16),
                                  jnp.bfloat16).astype(jnp.float32)
    hi = lax.bitcast_convert_type((w >> 16).astype(jnp.uint16),
                                  jnp.bfloat16).astype(jnp.float32)
    return lo, hi


def _msgT_packed(acc_relu):
    half = acc_relu.shape[0] // 2
    return _pack_pair(acc_relu[:half], acc_relu[half:])


def _tc_msg0_body(x_ref, w_ref, bcol_ref, s_ref, d_ref, o_ref, oe_ref):
    acc = lax.dot_general(w_ref[...], x_ref[...], _DN_T,
                          preferred_element_type=jnp.float32)
    acc = jnp.maximum(acc + bcol_ref[...], 0.0)
    o_ref[...] = _msgT_packed(acc)
    oe_ref[...] = s_ref[...] | (d_ref[...] << 16)


def _tc_msg0(x, w, b, src, dst, rows=2500):
    n = x.shape[0]
    d = w.shape[1]
    e = src.shape[0]
    msgP, epacked = pl.pallas_call(
        _tc_msg0_body,
        out_shape=(
            jax.ShapeDtypeStruct((d // 2, n), jnp.int32),
            jax.ShapeDtypeStruct((rows, e // rows), jnp.int32),
        ),
    )(x, w, b.reshape(d, 1),
      src.reshape(rows, e // rows), dst.reshape(rows, e // rows))
    return msgP, epacked.reshape(e)


def _bot_matmul(ap, wb):
    half = wb.shape[0] // 2
    age, ago = _unpack_pair(ap)
    return (lax.dot_general(age, wb[0:half], _TN_T,
                            preferred_element_type=jnp.float32)
            + lax.dot_general(ago, wb[half:], _TN_T,
                              preferred_element_type=jnp.float32))


def _tc_mid_body(h_ref, ap_ref, wfc_ref, bfc_ref, wp_ref, bpcol_ref,
                 h1_ref, m1_ref):
    din = h_ref.shape[1]
    top = jnp.dot(h_ref[...], wfc_ref[0:din],
                  preferred_element_type=jnp.float32)
    bot = _bot_matmul(ap_ref[...], wfc_ref[din:2 * din])
    h1 = jnp.maximum(top + bot + bfc_ref[...], 0.0)
    h1_ref[...] = h1
    macc = lax.dot_general(wp_ref[...], h1, _DN_T,
                           preferred_element_type=jnp.float32)
    macc = jnp.maximum(macc + bpcol_ref[...], 0.0)
    m1_ref[...] = _msgT_packed(macc)


def _tc_mid(h, aggP, wfc, bfc, wp, bp):
    n = h.shape[0]
    dout = wfc.shape[1]
    dp = wp.shape[1]
    return pl.pallas_call(
        _tc_mid_body,
        out_shape=(
            jax.ShapeDtypeStruct((n, dout), jnp.float32),
            jax.ShapeDtypeStruct((dp // 2, n), jnp.int32),
        ),
    )(h, aggP, wfc, bfc.reshape(1, dout), wp, bp.reshape(dp, 1))


def _tc_out_body(h_ref, ap_ref, wfc_ref, bfc_ref, o_ref):
    din = h_ref.shape[1]
    top = jnp.dot(h_ref[...], wfc_ref[0:din],
                  preferred_element_type=jnp.float32)
    bot = _bot_matmul(ap_ref[...], wfc_ref[din:2 * din])
    o_ref[...] = top + bot + bfc_ref[...]


def _tc_out(h, aggP, wfc, bfc):
    n = h.shape[0]
    dout = wfc.shape[1]
    return pl.pallas_call(
        _tc_out_body,
        out_shape=jax.ShapeDtypeStruct((n, dout), jnp.float32),
    )(h, aggP, wfc, bfc.reshape(1, dout))


def _pmax(cur_w, val_w):
    cur_b = plsc.bitcast(cur_w, jnp.bfloat16)
    val_b = plsc.bitcast(val_w, jnp.bfloat16)
    return plsc.bitcast(jnp.maximum(cur_b, val_b), jnp.int32)


def _sc_segmax_body(n, e, chunk,
                    msg_hbm, edges_hbm, agg_hbm,
                    m0, m1,
                    a00, a01, a10, a11, a20, a21, a30, a31,
                    eb0, eb1, sem0, sem1):
    c = lax.axis_index("c")
    s = lax.axis_index("s")
    wid = s * NC + c

    ubase, _ = plsc.scan_count(lax.iota(jnp.int32, 16))

    msgs = [m0, m1]
    aggs = [[a00, a01], [a10, a11], [a20, a21], [a30, a31]]
    ebufs = [(eb0, sem0), (eb1, sem1)]

    for f in range(2):
        pltpu.sync_copy(msg_hbm.at[wid, f], msgs[f])

    zero = jnp.zeros((LANES,), jnp.int32)

    def zbody(i, carry):
        for u in range(UNROLL):
            for f in range(2):
                aggs[u][f][pl.ds(i * LANES, LANES)] = zero
        return carry

    lax.fori_loop(0, n // LANES, zbody, 0)

    nchunks = e // chunk

    def start_fetch(ci, buf):
        eb, sem = buf
        pltpu.async_copy(edges_hbm.at[pl.ds(ci * chunk, chunk)], eb, sem)

    def drain(buf):
        eb, sem = buf
        pltpu.make_async_copy(edges_hbm.at[pl.ds(0, chunk)], eb, sem).wait()

    def process(buf):
        eb, _ = buf

        def vec_body(g, carry2):
            base = g * (UNROLL * LANES)
            ws = [eb[pl.ds(base + u * LANES, LANES)] for u in range(UNROLL)]
            svs = [w & 0xFFFF for w in ws]
            dvs = [lax.shift_right_logical(w, 16) for w in ws]

            valss = [[plsc.load_gather(msgs[f], [svs[u]]) for f in range(2)]
                     for u in range(UNROLL)]
            dupmask = None
            for u in range(UNROLL):
                counts, _ = plsc.scan_count(dvs[u])
                dup = counts > ubase
                dupmask = dup if dupmask is None else (dupmask | dup)
            curss = [[plsc.load_gather(aggs[u][f], [dvs[u]])
                      for f in range(2)] for u in range(UNROLL)]
            for u in range(UNROLL):
                for f in range(2):
                    new_w = _pmax(curss[u][f], valss[u][f])
                    plsc.store_scatter(aggs[u][f], [dvs[u]], new_w,
                                       mask=new_w != curss[u][f])

            ndup = plsc.all_reduce_population_count(dupmask)

            @pl.when(ndup[0] > 0)
            def _():
                for u in range(UNROLL):
                    for f in range(2):
                        def fix_body(_, u=u, f=f):
                            acc = aggs[u][f]
                            cur = plsc.load_gather(acc, [dvs[u]])
                            new_w = _pmax(cur, valss[u][f])
                            plsc.store_scatter(acc, [dvs[u]], new_w,
                                               mask=new_w != cur)
                            cur2 = plsc.load_gather(acc, [dvs[u]])
                            new2 = _pmax(cur2, valss[u][f])
                            pend = plsc.all_reduce_population_count(
                                new2 != cur2)
                            return pend[0] > 0

                        lax.while_loop(lambda keep: keep, fix_body,
                                       jnp.bool_(True))

            return carry2

        lax.fori_loop(0, chunk // (UNROLL * LANES), vec_body, 0)

    start_fetch(0, ebufs[0])

    def pair_body(i, carry):
        start_fetch(2 * i + 1, ebufs[1])
        drain(ebufs[0])
        process(ebufs[0])

        @pl.when(2 * i + 2 < nchunks)
        def _():
            start_fetch(2 * i + 2, ebufs[0])

        drain(ebufs[1])
        process(ebufs[1])
        return carry

    lax.fori_loop(0, nchunks // 2, pair_body, 0)

    def merge_body(i, carry):
        sl = pl.ds(i * LANES, LANES)
        for f in range(2):
            m01 = _pmax(aggs[0][f][sl], aggs[1][f][sl])
            m23 = _pmax(aggs[2][f][sl], aggs[3][f][sl])
            aggs[0][f][sl] = _pmax(m01, m23)
        return carry

    lax.fori_loop(0, n // LANES, merge_body, 0)

    for f in range(2):
        pltpu.sync_copy(aggs[0][f], agg_hbm.at[wid, f])


def _sc_segmax(msgP_t, epacked, chunk):
    nw, nf, n = msgP_t.shape
    e = epacked.shape[0]
    assert nw == NW and nf == 2 and e % (2 * chunk) == 0
    assert chunk % (UNROLL * LANES) == 0

    mesh = plsc.VectorSubcoreMesh(core_axis_name="c", subcore_axis_name="s",
                                  num_cores=NC, num_subcores=NS)
    kern = pl.kernel(
        functools.partial(_sc_segmax_body, n, e, chunk),
        out_type=jax.ShapeDtypeStruct((NW, 2, n), jnp.int32),
        mesh=mesh,
        compiler_params=pltpu.CompilerParams(needs_layout_passes=False),
        scratch_types=(
            [pltpu.VMEM((n,), jnp.int32) for _ in range(10)]
            + [pltpu.VMEM((chunk,), jnp.int32) for _ in range(2)]
            + [pltpu.SemaphoreType.DMA, pltpu.SemaphoreType.DMA]
        ),
    )
    return kern(msgP_t, epacked)


def kernel(x, edge_index, W_pool0, b_pool0, W_pool1, b_pool1,
           W_fc0, b_fc0, W_fc1, b_fc1):
    n, d = x.shape
    src = edge_index[0].astype(jnp.int32)
    dst = edge_index[1].astype(jnp.int32)
    chunk = 6400
    d1 = W_fc0.shape[1]

    msg0P, epacked = _tc_msg0(x, W_pool0, b_pool0, src, dst)
    agg0P = _sc_segmax(msg0P.reshape(NW, 2, n), epacked, chunk)

    h1, msg1P = _tc_mid(x, agg0P.reshape(d // 2, n), W_fc0, b_fc0,
                        W_pool1, b_pool1)

    agg1P = _sc_segmax(msg1P.reshape(NW, 2, n), epacked, chunk)

    return _tc_out(h1, agg1P.reshape(d1 // 2, n), W_fc1, b_fc1)


# speedup vs baseline: 7.9460x; 1.0114x over previous
import functools

import jax
import jax.numpy as jnp
from jax import lax
from jax.experimental import pallas as pl
from jax.experimental.pallas import tpu as pltpu
from jax.experimental.pallas import tpu_sc as plsc

NC = 2
NS = 16
NW = NC * NS
LANES = 16
UNROLL = 4

_DN_T = (((0,), (1,)), ((), ()))
_TN_T = (((0,), (0,)), ((), ()))


def _pack_pair(lo_f32, hi_f32):
    lo = lax.bitcast_convert_type(lo_f32.astype(jnp.bfloat16),
                                  jnp.uint16).astype(jnp.uint32)
    hi = lax.bitcast_convert_type(hi_f32.astype(jnp.bfloat16),
                                  jnp.uint16).astype(jnp.uint32)
    return lax.bitcast_convert_type(lo | (hi << 16), jnp.int32)


def _unpack_pair(packed):
    w = lax.bitcast_convert_type(packed, jnp.uint32)
    lo = lax.bitcast_convert_type((w & 0xFFFF).astype(jnp.uint16),
                                  jnp.bfloat16).astype(jnp.float32)
    hi = lax.bitcast_convert_type((w >> 16).astype(jnp.uint16),
                                  jnp.bfloat16).astype(jnp.float32)
    return lo, hi


def _msgT_packed(acc_relu):
    half = acc_relu.shape[0] // 2
    return _pack_pair(acc_relu[:half], acc_relu[half:])


def _tc_msg0_body(x_ref, w_ref, bcol_ref, s_ref, d_ref, o_ref, oe_ref):
    acc = lax.dot_general(w_ref[...], x_ref[...], _DN_T,
                          preferred_element_type=jnp.float32)
    acc = jnp.maximum(acc + bcol_ref[...], 0.0)
    o_ref[...] = _msgT_packed(acc)
    oe_ref[...] = s_ref[...] | (d_ref[...] << 16)


def _tc_msg0(x, w, b, src, dst, rows=2500):
    n = x.shape[0]
    d = w.shape[1]
    e = src.shape[0]
    msgP, epacked = pl.pallas_call(
        _tc_msg0_body,
        out_shape=(
            jax.ShapeDtypeStruct((d // 2, n), jnp.int32),
            jax.ShapeDtypeStruct((rows, e // rows), jnp.int32),
        ),
    )(x, w, b.reshape(d, 1),
      src.reshape(rows, e // rows), dst.reshape(rows, e // rows))
    return msgP, epacked.reshape(e)


def _bot_matmul(ap, wb):
    half = wb.shape[0] // 2
    age, ago = _unpack_pair(ap)
    return (lax.dot_general(age, wb[0:half], _TN_T,
                            preferred_element_type=jnp.float32)
            + lax.dot_general(ago, wb[half:], _TN_T,
                              preferred_element_type=jnp.float32))


def _tc_mid_body(h_ref, ap_ref, wfc_ref, bfc_ref, wp_ref, bpcol_ref,
                 h1_ref, m1_ref):
    din = h_ref.shape[1]
    top = jnp.dot(h_ref[...], wfc_ref[0:din],
                  preferred_element_type=jnp.float32)
    bot = _bot_matmul(ap_ref[...], wfc_ref[din:2 * din])
    h1 = jnp.maximum(top + bot + bfc_ref[...], 0.0)
    h1_ref[...] = h1
    macc = lax.dot_general(wp_ref[...], h1, _DN_T,
                           preferred_element_type=jnp.float32)
    macc = jnp.maximum(macc + bpcol_ref[...], 0.0)
    m1_ref[...] = _msgT_packed(macc)


def _tc_mid(h, aggP, wfc, bfc, wp, bp):
    n = h.shape[0]
    dout = wfc.shape[1]
    dp = wp.shape[1]
    return pl.pallas_call(
        _tc_mid_body,
        out_shape=(
            jax.ShapeDtypeStruct((n, dout), jnp.float32),
            jax.ShapeDtypeStruct((dp // 2, n), jnp.int32),
        ),
    )(h, aggP, wfc, bfc.reshape(1, dout), wp, bp.reshape(dp, 1))


def _tc_out_body(h_ref, ap_ref, wfc_ref, bfc_ref, o_ref):
    din = h_ref.shape[1]
    top = jnp.dot(h_ref[...], wfc_ref[0:din],
                  preferred_element_type=jnp.float32)
    bot = _bot_matmul(ap_ref[...], wfc_ref[din:2 * din])
    o_ref[...] = top + bot + bfc_ref[...]


def _tc_out(h, aggP, wfc, bfc):
    n = h.shape[0]
    dout = wfc.shape[1]
    return pl.pallas_call(
        _tc_out_body,
        out_shape=jax.ShapeDtypeStruct((n, dout), jnp.float32),
    )(h, aggP, wfc, bfc.reshape(1, dout))


def _pmax(cur_w, val_w):
    cur_b = plsc.bitcast(cur_w, jnp.bfloat16)
    val_b = plsc.bitcast(val_w, jnp.bfloat16)
    return plsc.bitcast(jnp.maximum(cur_b, val_b), jnp.int32)


def _sc_segmax_body(n, e, chunk,
                    msg_hbm, edges_hbm, agg_hbm,
                    m0, m1,
                    a00, a01, a10, a11, a20, a21, a30, a31,
                    eb0, eb1, sem0, sem1):
    c = lax.axis_index("c")
    s = lax.axis_index("s")
    wid = s * NC + c

    ubase, _ = plsc.scan_count(lax.iota(jnp.int32, 16))

    msgs = [m0, m1]
    aggs = [[a00, a01], [a10, a11], [a20, a21], [a30, a31]]
    ebufs = [(eb0, sem0), (eb1, sem1)]

    for f in range(2):
        pltpu.sync_copy(msg_hbm.at[wid, f], msgs[f])

    zero = jnp.zeros((LANES,), jnp.int32)

    def zbody(i, carry):
        for u in range(UNROLL):
            for f in range(2):
                aggs[u][f][pl.ds(i * LANES, LANES)] = zero
        return carry

    lax.fori_loop(0, n // LANES, zbody, 0)

    nchunks = e // chunk

    def start_fetch(ci, buf):
        eb, sem = buf
        pltpu.async_copy(edges_hbm.at[pl.ds(ci * chunk, chunk)], eb, sem)

    def drain(buf):
        eb, sem = buf
        pltpu.make_async_copy(edges_hbm.at[pl.ds(0, chunk)], eb, sem).wait()

    def process(buf):
        eb, _ = buf

        def vec_body(g, carry2):
            base = g * (UNROLL * LANES)
            ws = [eb[pl.ds(base + u * LANES, LANES)] for u in range(UNROLL)]
            svs = [w & 0xFFFF for w in ws]
            dvs = [lax.shift_right_logical(w, 16) for w in ws]

            dupmask = None
            for u in range(UNROLL):
                counts, _ = plsc.scan_count(dvs[u])
                dup = counts > ubase
                dupmask = dup if dupmask is None else (dupmask | dup)
            valss = [[plsc.load_gather(msgs[f], [svs[u]]) for f in range(2)]
                     for u in range(UNROLL)]
            curss = [[plsc.load_gather(aggs[u][f], [dvs[u]])
                      for f in range(2)] for u in range(UNROLL)]
            for u in range(UNROLL):
                for f in range(2):
                    new_w = _pmax(curss[u][f], valss[u][f])
                    plsc.store_scatter(aggs[u][f], [dvs[u]], new_w,
                                       mask=new_w != curss[u][f])

            ndup = plsc.all_reduce_population_count(dupmask)

            @pl.when(ndup[0] > 0)
            def _():
                for u in range(UNROLL):
                    for f in range(2):
                        def fix_body(_, u=u, f=f):
                            acc = aggs[u][f]
                            cur = plsc.load_gather(acc, [dvs[u]])
                            new_w = _pmax(cur, valss[u][f])
                            plsc.store_scatter(acc, [dvs[u]], new_w,
                                               mask=new_w != cur)
                            cur2 = plsc.load_gather(acc, [dvs[u]])
                            new2 = _pmax(cur2, valss[u][f])
                            pend = plsc.all_reduce_population_count(
                                new2 != cur2)
                            return pend[0] > 0

                        lax.while_loop(lambda keep: keep, fix_body,
                                       jnp.bool_(True))

            return carry2

        lax.fori_loop(0, chunk // (UNROLL * LANES), vec_body, 0)

    start_fetch(0, ebufs[0])

    def pair_body(i, carry):
        start_fetch(2 * i + 1, ebufs[1])
        drain(ebufs[0])
        process(ebufs[0])

        @pl.when(2 * i + 2 < nchunks)
        def _():
            start_fetch(2 * i + 2, ebufs[0])

        drain(ebufs[1])
        process(ebufs[1])
        return carry

    lax.fori_loop(0, nchunks // 2, pair_body, 0)

    def merge_body(i, carry):
        sl = pl.ds(i * LANES, LANES)
        for f in range(2):
            m01 = _pmax(aggs[0][f][sl], aggs[1][f][sl])
            m23 = _pmax(aggs[2][f][sl], aggs[3][f][sl])
            aggs[0][f][sl] = _pmax(m01, m23)
        return carry

    lax.fori_loop(0, n // LANES, merge_body, 0)

    for f in range(2):
        pltpu.sync_copy(aggs[0][f], agg_hbm.at[wid, f])


def _sc_segmax(msgP_t, epacked, chunk):
    nw, nf, n = msgP_t.shape
    e = epacked.shape[0]
    assert nw == NW and nf == 2 and e % (2 * chunk) == 0
    assert chunk % (UNROLL * LANES) == 0

    mesh = plsc.VectorSubcoreMesh(core_axis_name="c", subcore_axis_name="s",
                                  num_cores=NC, num_subcores=NS)
    kern = pl.kernel(
        functools.partial(_sc_segmax_body, n, e, chunk),
        out_type=jax.ShapeDtypeStruct((NW, 2, n), jnp.int32),
        mesh=mesh,
        compiler_params=pltpu.CompilerParams(needs_layout_passes=False),
        scratch_types=(
            [pltpu.VMEM((n,), jnp.int32) for _ in range(10)]
            + [pltpu.VMEM((chunk,), jnp.int32) for _ in range(2)]
            + [pltpu.SemaphoreType.DMA, pltpu.SemaphoreType.DMA]
        ),
    )
    return kern(msgP_t, epacked)


def kernel(x, edge_index, W_pool0, b_pool0, W_pool1, b_pool1,
           W_fc0, b_fc0, W_fc1, b_fc1):
    n, d = x.shape
    src = edge_index[0].astype(jnp.int32)
    dst = edge_index[1].astype(jnp.int32)
    chunk = 6400
    d1 = W_fc0.shape[1]

    msg0P, epacked = _tc_msg0(x, W_pool0, b_pool0, src, dst)
    agg0P = _sc_segmax(msg0P.reshape(NW, 2, n), epacked, chunk)

    h1, msg1P = _tc_mid(x, agg0P.reshape(d // 2, n), W_fc0, b_fc0,
                        W_pool1, b_pool1)

    agg1P = _sc_segmax(msg1P.reshape(NW, 2, n), epacked, chunk)

    return _tc_out(h1, agg1P.reshape(d1 // 2, n), W_fc1, b_fc1)


# speedup vs baseline: 8.7289x; 1.0985x over previous
import functools

import jax
import jax.numpy as jnp
from jax import lax
from jax.experimental import pallas as pl
from jax.experimental.pallas import tpu as pltpu
from jax.experimental.pallas import tpu_sc as plsc

NC = 2
NS = 16
NW = NC * NS
LANES = 16
UNROLL = 4

_DN_T = (((0,), (1,)), ((), ()))
_TN_T = (((0,), (0,)), ((), ()))


def _pack_pair(lo_f32, hi_f32):
    lo = lax.bitcast_convert_type(lo_f32.astype(jnp.bfloat16),
                                  jnp.uint16).astype(jnp.uint32)
    hi = lax.bitcast_convert_type(hi_f32.astype(jnp.bfloat16),
                                  jnp.uint16).astype(jnp.uint32)
    return lax.bitcast_convert_type(lo | (hi << 16), jnp.int32)


def _unpack_pair(packed):
    w = lax.bitcast_convert_type(packed, jnp.uint32)
    lo = lax.bitcast_convert_type((w & 0xFFFF).astype(jnp.uint16),
                                  jnp.bfloat16).astype(jnp.float32)
    hi = lax.bitcast_convert_type((w >> 16).astype(jnp.uint16),
                                  jnp.bfloat16).astype(jnp.float32)
    return lo, hi


def _msgT_packed(acc_relu):
    half = acc_relu.shape[0] // 2
    return _pack_pair(acc_relu[:half], acc_relu[half:])


def _tc_msg0_body(x_ref, w_ref, bcol_ref, s_ref, d_ref, o_ref, oe_ref):
    acc = lax.dot_general(w_ref[...], x_ref[...], _DN_T,
                          preferred_element_type=jnp.float32)
    acc = jnp.maximum(acc + bcol_ref[...], 0.0)
    o_ref[...] = _msgT_packed(acc)
    oe_ref[...] = s_ref[...] | (d_ref[...] << 16)


def _tc_msg0(x, w, b, src, dst, rows=2500):
    n = x.shape[0]
    d = w.shape[1]
    e = src.shape[0]
    msgP, epacked = pl.pallas_call(
        _tc_msg0_body,
        out_shape=(
            jax.ShapeDtypeStruct((d // 2, n), jnp.int32),
            jax.ShapeDtypeStruct((rows, e // rows), jnp.int32),
        ),
    )(x, w, b.reshape(d, 1),
      src.reshape(rows, e // rows), dst.reshape(rows, e // rows))
    return msgP, epacked.reshape(e)


def _bot_matmul(ap, wb):
    half = wb.shape[0] // 2
    age, ago = _unpack_pair(ap)
    return (lax.dot_general(age, wb[0:half], _TN_T,
                            preferred_element_type=jnp.float32)
            + lax.dot_general(ago, wb[half:], _TN_T,
                              preferred_element_type=jnp.float32))


def _tc_mid_body(h_ref, ap_ref, wfc_ref, bfc_ref, wp_ref, bpcol_ref,
                 h1_ref, m1_ref):
    din = h_ref.shape[1]
    top = jnp.dot(h_ref[...], wfc_ref[0:din],
                  preferred_element_type=jnp.float32)
    bot = _bot_matmul(ap_ref[...], wfc_ref[din:2 * din])
    h1 = jnp.maximum(top + bot + bfc_ref[...], 0.0)
    h1_ref[...] = h1
    macc = lax.dot_general(wp_ref[...], h1, _DN_T,
                           preferred_element_type=jnp.float32)
    macc = jnp.maximum(macc + bpcol_ref[...], 0.0)
    m1_ref[...] = _msgT_packed(macc)


def _tc_mid(h, aggP, wfc, bfc, wp, bp):
    n = h.shape[0]
    dout = wfc.shape[1]
    dp = wp.shape[1]
    return pl.pallas_call(
        _tc_mid_body,
        out_shape=(
            jax.ShapeDtypeStruct((n, dout), jnp.float32),
            jax.ShapeDtypeStruct((dp // 2, n), jnp.int32),
        ),
    )(h, aggP, wfc, bfc.reshape(1, dout), wp, bp.reshape(dp, 1))


def _tc_out_body(h_ref, ap_ref, wfc_ref, bfc_ref, o_ref):
    din = h_ref.shape[1]
    top = jnp.dot(h_ref[...], wfc_ref[0:din],
                  preferred_element_type=jnp.float32)
    bot = _bot_matmul(ap_ref[...], wfc_ref[din:2 * din])
    o_ref[...] = top + bot + bfc_ref[...]


def _tc_out(h, aggP, wfc, bfc):
    n = h.shape[0]
    dout = wfc.shape[1]
    return pl.pallas_call(
        _tc_out_body,
        out_shape=jax.ShapeDtypeStruct((n, dout), jnp.float32),
    )(h, aggP, wfc, bfc.reshape(1, dout))


def _pmax(cur_w, val_w):
    cur_b = plsc.bitcast(cur_w, jnp.bfloat16)
    val_b = plsc.bitcast(val_w, jnp.bfloat16)
    return plsc.bitcast(jnp.maximum(cur_b, val_b), jnp.int32)


def _sc_segmax_body(n, e, chunk,
                    msg_hbm, edges_hbm, agg_hbm,
                    m0, m1,
                    a00, a01, a10, a11, a20, a21, a30, a31,
                    eb0, eb1, sem0, sem1):
    c = lax.axis_index("c")
    s = lax.axis_index("s")
    wid = s * NC + c

    ubase, _ = plsc.scan_count(lax.iota(jnp.int32, 16))

    msgs = [m0, m1]
    aggs = [[a00, a01], [a10, a11], [a20, a21], [a30, a31]]
    ebufs = [(eb0, sem0), (eb1, sem1)]

    for f in range(2):
        pltpu.sync_copy(msg_hbm.at[wid, f], msgs[f])

    zero = jnp.zeros((LANES,), jnp.int32)

    def zbody(i, carry):
        for p in range(4):
            for f in range(2):
                aggs[p][f][pl.ds(i * LANES, LANES)] = zero
        return carry

    lax.fori_loop(0, n // LANES, zbody, 0)

    nchunks = e // chunk

    def start_fetch(ci, buf):
        eb, sem = buf
        pltpu.async_copy(edges_hbm.at[pl.ds(ci * chunk, chunk)], eb, sem)

    def drain(buf):
        eb, sem = buf
        pltpu.make_async_copy(edges_hbm.at[pl.ds(0, chunk)], eb, sem).wait()

    ubase1 = ubase + 1

    def process(buf):
        eb, _ = buf

        def load_group(g):
            base = g * (UNROLL * LANES)
            ws = [eb[pl.ds(base + u * LANES, LANES)] for u in range(UNROLL)]
            svs = [w & 0xFFFF for w in ws]
            dvs = [lax.shift_right_logical(w, 16) for w in ws]
            return svs, dvs

        def vec_body(g, dirty):
            svs, dvs = load_group(g)

            deep = None
            for u in range(UNROLL):
                counts, _ = plsc.scan_count(dvs[u])
                d = counts > ubase1
                deep = d if deep is None else (deep | d)
            valss = [[plsc.load_gather(msgs[f], [svs[u]]) for f in range(2)]
                     for u in range(UNROLL)]
            for _round in range(2):
                curss = [[plsc.load_gather(aggs[u][f], [dvs[u]])
                          for f in range(2)] for u in range(UNROLL)]
                for u in range(UNROLL):
                    for f in range(2):
                        new_w = _pmax(curss[u][f], valss[u][f])
                        plsc.store_scatter(aggs[u][f], [dvs[u]], new_w,
                                           mask=new_w != curss[u][f])

            return dirty | deep

        dirty = lax.fori_loop(0, chunk // (UNROLL * LANES), vec_body,
                              lax.full((LANES,), False))
        ndirty = plsc.all_reduce_population_count(dirty)

        @pl.when(ndirty[0] > 0)
        def _():
            def rep_body(g, carry):
                svs, dvs = load_group(g)
                for u in range(UNROLL):
                    for f in range(2):
                        val = plsc.load_gather(msgs[f], [svs[u]])

                        def fix_body(_, u=u, f=f, val=val):
                            acc = aggs[u][f]
                            cur = plsc.load_gather(acc, [dvs[u]])
                            new_w = _pmax(cur, val)
                            plsc.store_scatter(acc, [dvs[u]], new_w,
                                               mask=new_w != cur)
                            cur2 = plsc.load_gather(acc, [dvs[u]])
                            new2 = _pmax(cur2, val)
                            pend = plsc.all_reduce_population_count(
                                new2 != cur2)
                            return pend[0] > 0

                        lax.while_loop(lambda keep: keep, fix_body,
                                       jnp.bool_(True))
                return carry

            lax.fori_loop(0, chunk // (UNROLL * LANES), rep_body, 0)

    start_fetch(0, ebufs[0])

    def pair_body(i, carry):
        start_fetch(2 * i + 1, ebufs[1])
        drain(ebufs[0])
        process(ebufs[0])

        @pl.when(2 * i + 2 < nchunks)
        def _():
            start_fetch(2 * i + 2, ebufs[0])

        drain(ebufs[1])
        process(ebufs[1])
        return carry

    lax.fori_loop(0, nchunks // 2, pair_body, 0)

    def merge_body(i, carry):
        sl = pl.ds(i * LANES, LANES)
        for f in range(2):
            m01 = _pmax(aggs[0][f][sl], aggs[1][f][sl])
            m23 = _pmax(aggs[2][f][sl], aggs[3][f][sl])
            aggs[0][f][sl] = _pmax(m01, m23)
        return carry

    lax.fori_loop(0, n // LANES, merge_body, 0)

    for f in range(2):
        pltpu.sync_copy(aggs[0][f], agg_hbm.at[wid, f])


def _sc_segmax(msgP_t, epacked, chunk):
    nw, nf, n = msgP_t.shape
    e = epacked.shape[0]
    assert nw == NW and nf == 2 and e % (2 * chunk) == 0
    assert chunk % (UNROLL * LANES) == 0

    mesh = plsc.VectorSubcoreMesh(core_axis_name="c", subcore_axis_name="s",
                                  num_cores=NC, num_subcores=NS)
    kern = pl.kernel(
        functools.partial(_sc_segmax_body, n, e, chunk),
        out_type=jax.ShapeDtypeStruct((NW, 2, n), jnp.int32),
        mesh=mesh,
        compiler_params=pltpu.CompilerParams(needs_layout_passes=False),
        scratch_types=(
            [pltpu.VMEM((n,), jnp.int32) for _ in range(10)]
            + [pltpu.VMEM((chunk,), jnp.int32) for _ in range(2)]
            + [pltpu.SemaphoreType.DMA, pltpu.SemaphoreType.DMA]
        ),
    )
    return kern(msgP_t, epacked)


def kernel(x, edge_index, W_pool0, b_pool0, W_pool1, b_pool1,
           W_fc0, b_fc0, W_fc1, b_fc1):
    n, d = x.shape
    src = edge_index[0].astype(jnp.int32)
    dst = edge_index[1].astype(jnp.int32)
    chunk = 6400
    d1 = W_fc0.shape[1]

    msg0P, epacked = _tc_msg0(x, W_pool0, b_pool0, src, dst)
    agg0P = _sc_segmax(msg0P.reshape(NW, 2, n), epacked, chunk)

    h1, msg1P = _tc_mid(x, agg0P.reshape(d // 2, n), W_fc0, b_fc0,
                        W_pool1, b_pool1)

    agg1P = _sc_segmax(msg1P.reshape(NW, 2, n), epacked, chunk)

    return _tc_out(h1, agg1P.reshape(d1 // 2, n), W_fc1, b_fc1)
